# baseline (speedup 1.0000x reference)
import jax
import jax.numpy as jnp
from jax.experimental import pallas as pl

_LR = 0.01
_W_MIN = 0.0
_W_MAX = 1.0

_BR = 256


def _update_block(w_ref, pre_ref, post_ref, out_ref):
    dw = _LR * post_ref[...] * pre_ref[...]
    out_ref[...] = jnp.clip(w_ref[...] + dw, _W_MIN, _W_MAX)


def kernel(weights, pre, post):
    n_post, n_pre = weights.shape
    pre2 = pre.reshape(1, n_pre)
    post2 = post.reshape(n_post, 1)
    grid = (n_post // _BR,)
    return pl.pallas_call(
        _update_block,
        grid=grid,
        in_specs=[
            pl.BlockSpec((_BR, n_pre), lambda i: (i, 0)),
            pl.BlockSpec((1, n_pre), lambda i: (0, 0)),
            pl.BlockSpec((_BR, 1), lambda i: (i, 0)),
        ],
        out_specs=pl.BlockSpec((_BR, n_pre), lambda i: (i, 0)),
        out_shape=jax.ShapeDtypeStruct((n_post, n_pre), weights.dtype),
    )(weights, pre2, post2)


# speedup vs baseline: 1.0195x; 1.0195x over previous
import jax
import jax.numpy as jnp
from jax.experimental import pallas as pl

_LR = 0.01
_W_MIN = 0.0
_W_MAX = 1.0

_BR = 512
_RB = 8


def _update_block(w_ref, pre_ref, post_ref, out_ref):
    pre_row = pre_ref[...]
    for i in range(_BR // _RB):
        sl = pl.ds(i * _RB, _RB)
        dw = post_ref[sl, :] * pre_row
        out_ref[sl, :] = jnp.clip(w_ref[sl, :] + dw, _W_MIN, _W_MAX)


def kernel(weights, pre, post):
    n_post, n_pre = weights.shape
    pre2 = pre.reshape(1, n_pre)
    post2 = (_LR * post).reshape(n_post, 1)
    grid = (n_post // _BR,)
    return pl.pallas_call(
        _update_block,
        grid=grid,
        in_specs=[
            pl.BlockSpec((_BR, n_pre), lambda i: (i, 0)),
            pl.BlockSpec((1, n_pre), lambda i: (0, 0)),
            pl.BlockSpec((_BR, 1), lambda i: (i, 0)),
        ],
        out_specs=pl.BlockSpec((_BR, n_pre), lambda i: (i, 0)),
        out_shape=jax.ShapeDtypeStruct((n_post, n_pre), weights.dtype),
    )(weights, pre2, post2)
